# Initial kernel scaffold; baseline (speedup 1.0000x reference)
import jax
import jax.numpy as jnp
from jax.experimental import pallas as pl


def kernel(x, Ws1, bs1, W1, b1, Ws2, bs2, W2, b2):
    raise NotImplementedError("write your pallas kernel here")


# speedup vs baseline: 3.0002x; 3.0002x over previous
import functools

import jax
import jax.numpy as jnp
from jax import lax
from jax.experimental import pallas as pl

B = 64
NPATCH = 256
DIM = 384
HID = 1536
KEEP1 = 307
KEEP2 = 1228
K2PAD = 1280


def _masks(s_c, d, keep, chunk):
    f32 = jnp.float32
    s_r = jnp.transpose(s_c, (1, 0))
    nch = d // chunk
    rank_c = jnp.zeros((d, 1), f32)
    for jc in range(nch):
        s_rj = lax.slice(s_r, (0, jc * chunk), (1, (jc + 1) * chunk))
        il = lax.broadcasted_iota(jnp.int32, (d, chunk), 1) + jc * chunk
        isub = lax.broadcasted_iota(jnp.int32, (d, chunk), 0)
        cmp = (s_rj > s_c) | ((s_rj == s_c) & (il < isub))
        rank_c = rank_c + jnp.sum(cmp.astype(f32), axis=1, keepdims=True)
    kept_c = rank_c < keep
    kept_r = jnp.transpose(kept_c, (1, 0))
    pos_c = jnp.zeros((d, 1), f32)
    for jc in range(nch):
        k_rj = lax.slice(kept_r, (0, jc * chunk), (1, (jc + 1) * chunk))
        il = lax.broadcasted_iota(jnp.int32, (d, chunk), 1) + jc * chunk
        isub = lax.broadcasted_iota(jnp.int32, (d, chunk), 0)
        m = k_rj & (il < isub)
        pos_c = pos_c + jnp.sum(m.astype(f32), axis=1, keepdims=True)
    pos_c = pos_c.astype(jnp.int32)
    pos_r = jnp.transpose(pos_c, (1, 0))
    return kept_c, kept_r, pos_c, pos_r


def _body(x_ref, ws1c_ref, w1p_ref, b1c_ref, ws2c_ref, w2tp_ref, b2r_ref,
          out_ref):
    f32 = jnp.float32
    xb = x_ref[0]

    a1c = lax.dot_general(xb, ws1c_ref[...], (((0,), (0,)), ((), ())),
                          preferred_element_type=f32)
    kept_c, _, pos_c, _ = _masks(a1c, DIM, KEEP1, DIM)

    il = lax.broadcasted_iota(jnp.int32, (DIM, DIM), 1)
    p1 = jnp.where(kept_c & (pos_c == il), 1.0, 0.0).astype(f32)
    xc = lax.dot_general(xb, p1, (((1,), (0,)), ((), ())),
                         preferred_element_type=f32)

    hT = lax.dot_general(w1p_ref[...], xc, (((1,), (1,)), ((), ())),
                         preferred_element_type=f32)
    hT = jnp.maximum(hT + b1c_ref[...], 0.0)

    a2c = lax.dot_general(hT, ws2c_ref[...], (((1,), (0,)), ((), ())),
                          preferred_element_type=f32)
    _, kept2_r, _, pos2_r = _masks(a2c, HID, KEEP2, 512)

    acc = jnp.zeros((NPATCH, DIM), f32)
    for kb in range(K2PAD // 256):
        isub = lax.broadcasted_iota(jnp.int32, (256, HID), 0) + (kb * 256)
        p2t = jnp.where(kept2_r & (pos2_r == isub), 1.0, 0.0).astype(f32)
        hc = lax.dot_general(p2t, hT, (((1,), (0,)), ((), ())),
                             preferred_element_type=f32)
        w2b = lax.slice(w2tp_ref[...], (kb * 256, 0), ((kb + 1) * 256, DIM))
        acc = acc + lax.dot_general(hc, w2b, (((0,), (0,)), ((), ())),
                                    preferred_element_type=f32)
    out_ref[0] = acc + b2r_ref[...]


@functools.partial(jax.jit, static_argnames=("interpret",))
def kernel(x, Ws1, bs1, W1, b1, Ws2, bs2, W2, b2, interpret=False):
    f32 = jnp.float32
    ws1c = Ws1.reshape(NPATCH, 1).astype(f32)
    w1p = jnp.pad(W1, ((0, 0), (0, DIM - KEEP1))).astype(f32)
    b1c = b1.reshape(HID, 1).astype(f32)
    ws2c = Ws2.reshape(NPATCH, 1).astype(f32)
    w2tp = jnp.pad(W2.T, ((0, K2PAD - KEEP2), (0, 0))).astype(f32)
    b2r = b2.reshape(1, DIM).astype(f32)

    out = pl.pallas_call(
        _body,
        grid=(B,),
        in_specs=[
            pl.BlockSpec((1, NPATCH, DIM), lambda b: (b, 0, 0)),
            pl.BlockSpec((NPATCH, 1), lambda b: (0, 0)),
            pl.BlockSpec((HID, DIM), lambda b: (0, 0)),
            pl.BlockSpec((HID, 1), lambda b: (0, 0)),
            pl.BlockSpec((NPATCH, 1), lambda b: (0, 0)),
            pl.BlockSpec((K2PAD, DIM), lambda b: (0, 0)),
            pl.BlockSpec((1, DIM), lambda b: (0, 0)),
        ],
        out_specs=pl.BlockSpec((1, NPATCH, DIM), lambda b: (b, 0, 0)),
        out_shape=jax.ShapeDtypeStruct((B, NPATCH, DIM), f32),
        interpret=interpret,
    )(x, ws1c, w1p, b1c, ws2c, w2tp, b2r)
    return out


# speedup vs baseline: 3.6330x; 1.2109x over previous
import functools

import jax
import jax.numpy as jnp
from jax import lax
from jax.experimental import pallas as pl
from jax.experimental.pallas import tpu as pltpu

B = 64
NPATCH = 256
DIM = 384
HID = 1536
KEEP1 = 307
KEEP2 = 1228
K2PAD = 1280


def _masks(s_c, d, keep, chunk):
    f32 = jnp.float32
    s_r = jnp.transpose(s_c, (1, 0))
    nch = d // chunk
    rank_c = jnp.zeros((d, 1), f32)
    for jc in range(nch):
        s_rj = lax.slice(s_r, (0, jc * chunk), (1, (jc + 1) * chunk))
        il = lax.broadcasted_iota(jnp.int32, (d, chunk), 1) + jc * chunk
        isub = lax.broadcasted_iota(jnp.int32, (d, chunk), 0)
        cmp = (s_rj > s_c) | ((s_rj == s_c) & (il < isub))
        rank_c = rank_c + jnp.sum(cmp.astype(f32), axis=1, keepdims=True)
    kept_c = rank_c < keep
    kept_r = jnp.transpose(kept_c, (1, 0))
    pos_c = jnp.zeros((d, 1), f32)
    for jc in range(nch):
        k_rj = lax.slice(kept_r, (0, jc * chunk), (1, (jc + 1) * chunk))
        il = lax.broadcasted_iota(jnp.int32, (d, chunk), 1) + jc * chunk
        isub = lax.broadcasted_iota(jnp.int32, (d, chunk), 0)
        m = k_rj & (il < isub)
        pos_c = pos_c + jnp.sum(m.astype(f32), axis=1, keepdims=True)
    pos_c = pos_c.astype(jnp.int32)
    pos_r = jnp.transpose(pos_c, (1, 0))
    return kept_c, kept_r, pos_c, pos_r


def _body(x_ref, ws1c_ref, w1p_ref, b1c_ref, ws2c_ref, w2tp_ref, b2r_ref,
          out_ref, hc_ref):
    f32 = jnp.float32
    xb = x_ref[0]

    a1c = lax.dot_general(xb, ws1c_ref[...], (((0,), (0,)), ((), ())),
                          preferred_element_type=f32)
    kept_c, _, pos_c, _ = _masks(a1c, DIM, KEEP1, DIM)

    il = lax.broadcasted_iota(jnp.int32, (DIM, DIM), 1)
    p1 = jnp.where(kept_c & (pos_c == il), 1.0, 0.0).astype(f32)
    xc = lax.dot_general(xb, p1, (((1,), (0,)), ((), ())),
                         preferred_element_type=f32)

    hT = lax.dot_general(w1p_ref[...], xc, (((1,), (1,)), ((), ())),
                         preferred_element_type=f32)
    hT = jnp.maximum(hT + b1c_ref[...], 0.0)

    a2c = lax.dot_general(hT, ws2c_ref[...], (((1,), (0,)), ((), ())),
                          preferred_element_type=f32)
    _, kept2_r, _, pos2_r = _masks(a2c, HID, KEEP2, 512)

    hc_ref[...] = jnp.zeros((HID, NPATCH), f32)
    for sb in range(HID // 256):
        w0 = max(0, sb * 256 - 320)
        k2b = lax.slice(kept2_r, (0, sb * 256), (1, (sb + 1) * 256))
        p2b = lax.slice(pos2_r, (0, sb * 256), (1, (sb + 1) * 256))
        isub = lax.broadcasted_iota(jnp.int32, (576, 256), 0) + w0
        p2t = jnp.where(k2b & (p2b == isub), 1.0, 0.0).astype(f32)
        hblk = lax.slice(hT, (sb * 256, 0), ((sb + 1) * 256, NPATCH))
        contrib = lax.dot_general(p2t, hblk, (((1,), (0,)), ((), ())),
                                  preferred_element_type=f32)
        hc_ref[w0:w0 + 576, :] = hc_ref[w0:w0 + 576, :] + contrib
    hc = hc_ref[0:K2PAD, :]
    out = lax.dot_general(hc, w2tp_ref[...], (((0,), (0,)), ((), ())),
                          preferred_element_type=f32)
    out_ref[0] = out + b2r_ref[...]


@functools.partial(jax.jit, static_argnames=("interpret",))
def kernel(x, Ws1, bs1, W1, b1, Ws2, bs2, W2, b2, interpret=False):
    f32 = jnp.float32
    ws1c = Ws1.reshape(NPATCH, 1).astype(f32)
    w1p = jnp.pad(W1, ((0, 0), (0, DIM - KEEP1))).astype(f32)
    b1c = b1.reshape(HID, 1).astype(f32)
    ws2c = Ws2.reshape(NPATCH, 1).astype(f32)
    w2tp = jnp.pad(W2.T, ((0, K2PAD - KEEP2), (0, 0))).astype(f32)
    b2r = b2.reshape(1, DIM).astype(f32)

    out = pl.pallas_call(
        _body,
        grid=(B,),
        in_specs=[
            pl.BlockSpec((1, NPATCH, DIM), lambda b: (b, 0, 0)),
            pl.BlockSpec((NPATCH, 1), lambda b: (0, 0)),
            pl.BlockSpec((HID, DIM), lambda b: (0, 0)),
            pl.BlockSpec((HID, 1), lambda b: (0, 0)),
            pl.BlockSpec((NPATCH, 1), lambda b: (0, 0)),
            pl.BlockSpec((K2PAD, DIM), lambda b: (0, 0)),
            pl.BlockSpec((1, DIM), lambda b: (0, 0)),
        ],
        out_specs=pl.BlockSpec((1, NPATCH, DIM), lambda b: (b, 0, 0)),
        out_shape=jax.ShapeDtypeStruct((B, NPATCH, DIM), f32),
        scratch_shapes=[pltpu.VMEM((HID, NPATCH), f32)],
        interpret=interpret,
    )(x, ws1c, w1p, b1c, ws2c, w2tp, b2r)
    return out


# speedup vs baseline: 3.7131x; 1.0221x over previous
import functools

import jax
import jax.numpy as jnp
from jax import lax
from jax.experimental import pallas as pl
from jax.experimental.pallas import tpu as pltpu

B = 64
NPATCH = 256
DIM = 384
HID = 1536
KEEP1 = 307
KEEP2 = 1228
K2PAD = 1280


def _masks(s_c, d, keep, chunk):
    f32 = jnp.float32
    s_r = jnp.transpose(s_c, (1, 0))
    nch = d // chunk
    rank_c = jnp.zeros((d, 1), f32)
    for jc in range(nch):
        s_rj = lax.slice(s_r, (0, jc * chunk), (1, (jc + 1) * chunk))
        il = lax.broadcasted_iota(jnp.int32, (d, chunk), 1) + jc * chunk
        isub = lax.broadcasted_iota(jnp.int32, (d, chunk), 0)
        cmp = (s_rj > s_c) | ((s_rj == s_c) & (il < isub))
        rank_c = rank_c + jnp.sum(cmp.astype(f32), axis=1, keepdims=True)
    kept_c = rank_c < keep
    kept_r = jnp.transpose(kept_c, (1, 0))
    pos_c = jnp.zeros((d, 1), f32)
    for jc in range(nch):
        k_rj = lax.slice(kept_r, (0, jc * chunk), (1, (jc + 1) * chunk))
        il = lax.broadcasted_iota(jnp.int32, (d, chunk), 1) + jc * chunk
        isub = lax.broadcasted_iota(jnp.int32, (d, chunk), 0)
        m = k_rj & (il < isub)
        pos_c = pos_c + jnp.sum(m.astype(f32), axis=1, keepdims=True)
    pos_c = pos_c.astype(jnp.int32)
    pos_r = jnp.transpose(pos_c, (1, 0))
    return kept_c, kept_r, pos_c, pos_r


PAIR = 2


def _body(x_ref, ws1c_ref, w1p_ref, b1c_ref, ws2c_ref, w2tp_ref, b2r_ref,
          out_ref, hc_ref):
    for i in range(PAIR):
        _one_sample(i, x_ref, ws1c_ref, w1p_ref, b1c_ref, ws2c_ref,
                    w2tp_ref, b2r_ref, out_ref, hc_ref)


def _one_sample(i, x_ref, ws1c_ref, w1p_ref, b1c_ref, ws2c_ref, w2tp_ref,
                b2r_ref, out_ref, hc_ref):
    f32 = jnp.float32
    xb = x_ref[i]

    a1c = lax.dot_general(xb, ws1c_ref[...], (((0,), (0,)), ((), ())),
                          preferred_element_type=f32)
    kept_c, _, pos_c, _ = _masks(a1c, DIM, KEEP1, DIM)

    il = lax.broadcasted_iota(jnp.int32, (DIM, DIM), 1)
    p1 = jnp.where(kept_c & (pos_c == il), 1.0, 0.0).astype(f32)
    xc = lax.dot_general(xb, p1, (((1,), (0,)), ((), ())),
                         preferred_element_type=f32)

    hT = lax.dot_general(w1p_ref[...], xc, (((1,), (1,)), ((), ())),
                         preferred_element_type=f32)
    hT = jnp.maximum(hT + b1c_ref[...], 0.0)

    a2c = lax.dot_general(hT, ws2c_ref[...], (((1,), (0,)), ((), ())),
                          preferred_element_type=f32)
    _, kept2_r, _, pos2_r = _masks(a2c, HID, KEEP2, 512)

    hc_ref[i] = jnp.zeros((HID, NPATCH), f32)
    for sb in range(HID // 256):
        w0 = max(0, sb * 256 - 320)
        k2b = lax.slice(kept2_r, (0, sb * 256), (1, (sb + 1) * 256))
        p2b = lax.slice(pos2_r, (0, sb * 256), (1, (sb + 1) * 256))
        isub = lax.broadcasted_iota(jnp.int32, (576, 256), 0) + w0
        p2t = jnp.where(k2b & (p2b == isub), 1.0, 0.0).astype(f32)
        hblk = lax.slice(hT, (sb * 256, 0), ((sb + 1) * 256, NPATCH))
        contrib = lax.dot_general(p2t, hblk, (((1,), (0,)), ((), ())),
                                  preferred_element_type=f32)
        hc_ref[i, w0:w0 + 576, :] = hc_ref[i, w0:w0 + 576, :] + contrib
    hc = hc_ref[i, 0:K2PAD, :]
    out = lax.dot_general(hc, w2tp_ref[...], (((0,), (0,)), ((), ())),
                          preferred_element_type=f32)
    out_ref[i] = out + b2r_ref[...]


@functools.partial(jax.jit, static_argnames=("interpret",))
def kernel(x, Ws1, bs1, W1, b1, Ws2, bs2, W2, b2, interpret=False):
    f32 = jnp.float32
    ws1c = Ws1.reshape(NPATCH, 1).astype(f32)
    w1p = jnp.pad(W1, ((0, 0), (0, DIM - KEEP1))).astype(f32)
    b1c = b1.reshape(HID, 1).astype(f32)
    ws2c = Ws2.reshape(NPATCH, 1).astype(f32)
    w2tp = jnp.pad(W2.T, ((0, K2PAD - KEEP2), (0, 0))).astype(f32)
    b2r = b2.reshape(1, DIM).astype(f32)

    out = pl.pallas_call(
        _body,
        grid=(B // PAIR,),
        in_specs=[
            pl.BlockSpec((PAIR, NPATCH, DIM), lambda b: (b, 0, 0)),
            pl.BlockSpec((NPATCH, 1), lambda b: (0, 0)),
            pl.BlockSpec((HID, DIM), lambda b: (0, 0)),
            pl.BlockSpec((HID, 1), lambda b: (0, 0)),
            pl.BlockSpec((NPATCH, 1), lambda b: (0, 0)),
            pl.BlockSpec((K2PAD, DIM), lambda b: (0, 0)),
            pl.BlockSpec((1, DIM), lambda b: (0, 0)),
        ],
        out_specs=pl.BlockSpec((PAIR, NPATCH, DIM), lambda b: (b, 0, 0)),
        out_shape=jax.ShapeDtypeStruct((B, NPATCH, DIM), f32),
        scratch_shapes=[pltpu.VMEM((PAIR, HID, NPATCH), f32)],
        interpret=interpret,
    )(x, ws1c, w1p, b1c, ws2c, w2tp, b2r)
    return out


# speedup vs baseline: 4.7547x; 1.2805x over previous
import functools

import jax
import jax.numpy as jnp
from jax import lax
from jax.experimental import pallas as pl
from jax.experimental.pallas import tpu as pltpu

B = 64
NPATCH = 256
DIM = 384
HID = 1536
KEEP1 = 307
KEEP2 = 1228
K2PAD = 1280


def _masks(s_c, d, keep, chunk):
    f32 = jnp.float32
    s_r = jnp.transpose(s_c, (1, 0))
    nch = d // chunk
    rank_r = jnp.zeros((1, d), f32)
    for ic in range(nch):
        s_ci = lax.slice(s_c, (ic * chunk, 0), ((ic + 1) * chunk, 1))
        il = lax.broadcasted_iota(jnp.int32, (chunk, d), 1)
        isub = lax.broadcasted_iota(jnp.int32, (chunk, d), 0) + ic * chunk
        cmp = (s_ci > s_r) | ((s_ci == s_r) & (isub < il))
        rank_r = rank_r + jnp.sum(cmp.astype(f32), axis=0, keepdims=True)
    kept_r = rank_r < keep
    kept_c = jnp.transpose(kept_r, (1, 0))
    pos_r = jnp.zeros((1, d), f32)
    for ic in range(nch):
        k_ci = lax.slice(kept_c, (ic * chunk, 0), ((ic + 1) * chunk, 1))
        il = lax.broadcasted_iota(jnp.int32, (chunk, d), 1)
        isub = lax.broadcasted_iota(jnp.int32, (chunk, d), 0) + ic * chunk
        m = k_ci & (isub < il)
        pos_r = pos_r + jnp.sum(m.astype(f32), axis=0, keepdims=True)
    pos_r = pos_r.astype(jnp.int32)
    return kept_r, pos_r


PAIR = 2


def _body(x_ref, ws1c_ref, w1p_ref, b1c_ref, ws2c_ref, w2tp_ref, b2r_ref,
          out_ref, hc_ref):
    for i in range(PAIR):
        _one_sample(i, x_ref, ws1c_ref, w1p_ref, b1c_ref, ws2c_ref,
                    w2tp_ref, b2r_ref, out_ref, hc_ref)


def _one_sample(i, x_ref, ws1c_ref, w1p_ref, b1c_ref, ws2c_ref, w2tp_ref,
                b2r_ref, out_ref, hc_ref):
    f32 = jnp.float32
    xb = x_ref[i]

    a1c = lax.dot_general(xb, ws1c_ref[...], (((0,), (0,)), ((), ())),
                          preferred_element_type=f32)
    kept1_r, pos1_r = _masks(a1c, DIM, KEEP1, DIM)

    isub = lax.broadcasted_iota(jnp.int32, (DIM, DIM), 0)
    p1t = jnp.where(kept1_r & (pos1_r == isub), 1.0, 0.0).astype(f32)
    xc = lax.dot_general(xb, p1t, (((1,), (1,)), ((), ())),
                         preferred_element_type=f32)

    hT = lax.dot_general(w1p_ref[...], xc, (((1,), (1,)), ((), ())),
                         preferred_element_type=f32)
    hT = jnp.maximum(hT + b1c_ref[...], 0.0)

    a2c = lax.dot_general(hT, ws2c_ref[...], (((1,), (0,)), ((), ())),
                          preferred_element_type=f32)
    kept2_r, pos2_r = _masks(a2c, HID, KEEP2, 512)

    hc_ref[i] = jnp.zeros((HID, NPATCH), f32)
    for sb in range(HID // 256):
        w0 = max(0, sb * 256 - 320)
        k2b = lax.slice(kept2_r, (0, sb * 256), (1, (sb + 1) * 256))
        p2b = lax.slice(pos2_r, (0, sb * 256), (1, (sb + 1) * 256))
        isub = lax.broadcasted_iota(jnp.int32, (576, 256), 0) + w0
        p2t = jnp.where(k2b & (p2b == isub), 1.0, 0.0).astype(f32)
        hblk = lax.slice(hT, (sb * 256, 0), ((sb + 1) * 256, NPATCH))
        contrib = lax.dot_general(p2t, hblk, (((1,), (0,)), ((), ())),
                                  preferred_element_type=f32)
        hc_ref[i, w0:w0 + 576, :] = hc_ref[i, w0:w0 + 576, :] + contrib
    hc = hc_ref[i, 0:K2PAD, :]
    out = lax.dot_general(hc, w2tp_ref[...], (((0,), (0,)), ((), ())),
                          preferred_element_type=f32)
    out_ref[i] = out + b2r_ref[...]


@functools.partial(jax.jit, static_argnames=("interpret",))
def kernel(x, Ws1, bs1, W1, b1, Ws2, bs2, W2, b2, interpret=False):
    f32 = jnp.float32
    ws1c = Ws1.reshape(NPATCH, 1).astype(f32)
    w1p = jnp.pad(W1, ((0, 0), (0, DIM - KEEP1))).astype(f32)
    b1c = b1.reshape(HID, 1).astype(f32)
    ws2c = Ws2.reshape(NPATCH, 1).astype(f32)
    w2tp = jnp.pad(W2.T, ((0, K2PAD - KEEP2), (0, 0))).astype(f32)
    b2r = b2.reshape(1, DIM).astype(f32)

    out = pl.pallas_call(
        _body,
        grid=(B // PAIR,),
        in_specs=[
            pl.BlockSpec((PAIR, NPATCH, DIM), lambda b: (b, 0, 0)),
            pl.BlockSpec((NPATCH, 1), lambda b: (0, 0)),
            pl.BlockSpec((HID, DIM), lambda b: (0, 0)),
            pl.BlockSpec((HID, 1), lambda b: (0, 0)),
            pl.BlockSpec((NPATCH, 1), lambda b: (0, 0)),
            pl.BlockSpec((K2PAD, DIM), lambda b: (0, 0)),
            pl.BlockSpec((1, DIM), lambda b: (0, 0)),
        ],
        out_specs=pl.BlockSpec((PAIR, NPATCH, DIM), lambda b: (b, 0, 0)),
        out_shape=jax.ShapeDtypeStruct((B, NPATCH, DIM), f32),
        scratch_shapes=[pltpu.VMEM((PAIR, HID, NPATCH), f32)],
        interpret=interpret,
    )(x, ws1c, w1p, b1c, ws2c, w2tp, b2r)
    return out
